# Initial kernel scaffold; baseline (speedup 1.0000x reference)
import functools

import jax
import jax.numpy as jnp
from jax import lax
from jax.experimental import pallas as pl
from jax.experimental.pallas import tpu as pltpu
from jax.experimental.pallas import tpu_sc as plsc

N_USER = 10000
N_GROUP = 10000
N_EDGE = 320000
D = 128
HALF = 64
NC = 2
NS = 16
L = 16

EDGES_PER_TILE = N_EDGE // NS
K = 80
NCH = EDGES_PER_TILE // K
ROWS_PER_TILE = N_GROUP // NS


def _body(hu2, srcr, dstr, rst_out, bsrc_out,
          src_v, dst_v, gidx_v, gbuf, ones_v, nbuf, degbuf,
          rst_acc, bsrc_acc, deg_dst, deg_src, sem):
    cid = lax.axis_index("c")
    sid = lax.axis_index("s")
    r0 = sid * ROWS_PER_TILE

    pltpu.sync_copy(srcr.at[sid], src_v)
    pltpu.sync_copy(dstr.at[sid], dst_v)

    one16 = jnp.full((L,), 1.0, jnp.float32)
    zero16 = jnp.zeros((L,), jnp.float32)

    def _fill(r, _):
        ones_v[r, :] = one16
        return 0
    lax.fori_loop(0, K, _fill, 0)

    def _zrow(r, _):
        for c in range(HALF // L):
            nbuf[r, pl.ds(c * L, L)] = zero16
        degbuf[r, :] = zero16
        return 0
    lax.fori_loop(0, ROWS_PER_TILE, _zrow, 0)

    pltpu.sync_copy(nbuf, rst_acc.at[pl.ds(r0, ROWS_PER_TILE)])
    pltpu.sync_copy(nbuf, bsrc_acc.at[pl.ds(r0, ROWS_PER_TILE)])
    pltpu.sync_copy(degbuf, deg_dst.at[pl.ds(r0, ROWS_PER_TILE)])
    pltpu.sync_copy(degbuf, deg_src.at[pl.ds(r0, ROWS_PER_TILE)])
    plsc.subcore_barrier()

    def _p1(j, _):
        for k in range(K // L):
            s16 = src_v[j, pl.ds(k * L, L)]
            gidx_v[pl.ds(k * L, L)] = s16 * 2 + cid
        pltpu.async_copy(hu2.at[gidx_v], gbuf, sem).wait()
        pltpu.sync_copy(gbuf, rst_acc.at[dst_v.at[j]], add=True)
        pltpu.sync_copy(ones_v, deg_dst.at[dst_v.at[j]], add=True)
        pltpu.sync_copy(ones_v, deg_src.at[src_v.at[j]], add=True)
        return 0
    lax.fori_loop(0, NCH, _p1, 0)
    plsc.subcore_barrier()

    pltpu.sync_copy(rst_acc.at[pl.ds(r0, ROWS_PER_TILE)], nbuf)
    pltpu.sync_copy(deg_dst.at[pl.ds(r0, ROWS_PER_TILE)], degbuf)

    def _nrm(r, _):
        scale = 1.0 / jnp.maximum(degbuf[r, :], 1.0)
        for c in range(HALF // L):
            nbuf[r, pl.ds(c * L, L)] = nbuf[r, pl.ds(c * L, L)] * scale
        return 0
    lax.fori_loop(0, ROWS_PER_TILE, _nrm, 0)

    pltpu.sync_copy(nbuf, rst_acc.at[pl.ds(r0, ROWS_PER_TILE)])
    pltpu.sync_copy(nbuf, rst_out.at[pl.ds(r0, ROWS_PER_TILE),
                                     pl.ds(cid * HALF, HALF)])
    plsc.subcore_barrier()

    def _p2(j, _):
        pltpu.async_copy(rst_acc.at[dst_v.at[j]], gbuf, sem).wait()
        pltpu.sync_copy(gbuf, bsrc_acc.at[src_v.at[j]], add=True)
        return 0
    lax.fori_loop(0, NCH, _p2, 0)
    plsc.subcore_barrier()

    pltpu.sync_copy(bsrc_acc.at[pl.ds(r0, ROWS_PER_TILE)], nbuf)
    pltpu.sync_copy(deg_src.at[pl.ds(r0, ROWS_PER_TILE)], degbuf)
    lax.fori_loop(0, ROWS_PER_TILE, _nrm, 0)
    pltpu.sync_copy(nbuf, bsrc_out.at[pl.ds(r0, ROWS_PER_TILE),
                                      pl.ds(cid * HALF, HALF)])


@jax.jit
def _lightgcn_sc(hu2, srcr, dstr):
    mesh = plsc.VectorSubcoreMesh(core_axis_name="c", subcore_axis_name="s",
                                  num_cores=NC, num_subcores=NS)
    f = pl.kernel(
        _body,
        out_type=[
            jax.ShapeDtypeStruct((N_GROUP, D), jnp.float32),
            jax.ShapeDtypeStruct((N_USER, D), jnp.float32),
        ],
        mesh=mesh,
        scratch_types=[
            pltpu.VMEM((NCH, K), jnp.int32),
            pltpu.VMEM((NCH, K), jnp.int32),
            pltpu.VMEM((K,), jnp.int32),
            pltpu.VMEM((K, HALF), jnp.float32),
            pltpu.VMEM((K, L), jnp.float32),
            pltpu.VMEM((ROWS_PER_TILE, HALF), jnp.float32),
            pltpu.VMEM((ROWS_PER_TILE, L), jnp.float32),
            pltpu.VMEM_SHARED((N_GROUP, HALF), jnp.float32),
            pltpu.VMEM_SHARED((N_USER, HALF), jnp.float32),
            pltpu.VMEM_SHARED((N_GROUP, L), jnp.float32),
            pltpu.VMEM_SHARED((N_USER, L), jnp.float32),
            pltpu.SemaphoreType.DMA,
        ],
    )
    return f(hu2, srcr, dstr)


def kernel(h_user, h_group, edge_index):
    del h_group
    hu2 = h_user.reshape(2 * N_USER, HALF)
    src = edge_index[0].astype(jnp.int32).reshape(NS, NCH, K)
    dst = edge_index[1].astype(jnp.int32).reshape(NS, NCH, K)
    rst, bsrc = _lightgcn_sc(hu2, src, dst)
    return (bsrc, rst)


# speedup vs baseline: 4.7570x; 4.7570x over previous
import jax
import jax.numpy as jnp
from jax import lax
from jax.experimental import pallas as pl
from jax.experimental.pallas import tpu as pltpu
from jax.experimental.pallas import tpu_sc as plsc

N_USER = 10000
N_GROUP = 10000
N_EDGE = 320000
D = 128
Q = 32
NQ = D // Q
NC = 2
NS = 16
L = 16

EDGES_PER_TILE = N_EDGE // NS
K = 80
NCH = EDGES_PER_TILE // K
ROWS_PER_TILE = N_GROUP // NS


def _gcn_pass(feat, gathr, scatr, out,
              gath_v, scat_v, gidx_a, gidx_b, gbuf_a, gbuf_b,
              ones_v, nbuf, degbuf, acc, deg, sem_a, sem_b):
    cid = lax.axis_index("c")
    sid = lax.axis_index("s")
    r0 = sid * ROWS_PER_TILE

    pltpu.sync_copy(gathr.at[sid], gath_v)
    pltpu.sync_copy(scatr.at[sid], scat_v)

    one16 = jnp.full((L,), 1.0, jnp.float32)
    zero16 = jnp.zeros((L,), jnp.float32)

    def _fill(r, _):
        ones_v[r, :] = one16
        return 0
    lax.fori_loop(0, K, _fill, 0)

    def _zacc(r, _):
        for c in range(Q // L):
            nbuf[r, pl.ds(c * L, L)] = zero16
        return 0

    def _zdeg(r, _):
        degbuf[r, :] = zero16
        return 0
    lax.fori_loop(0, ROWS_PER_TILE, _zdeg, 0)
    pltpu.sync_copy(degbuf, deg.at[pl.ds(r0, ROWS_PER_TILE)])

    for p in range(NQ // NC):
        qv = cid * (NQ // NC) + p

        lax.fori_loop(0, ROWS_PER_TILE, _zacc, 0)
        pltpu.sync_copy(nbuf, acc.at[pl.ds(r0, ROWS_PER_TILE)])
        plsc.subcore_barrier()

        def _chunk(j2, _):
            ja = j2 * 2
            jb = ja + 1
            for k in range(K // L):
                g16 = gath_v[ja, pl.ds(k * L, L)]
                gidx_a[pl.ds(k * L, L)] = g16 * NQ + qv
            cpa = pltpu.async_copy(feat.at[gidx_a], gbuf_a, sem_a)
            for k in range(K // L):
                g16 = gath_v[jb, pl.ds(k * L, L)]
                gidx_b[pl.ds(k * L, L)] = g16 * NQ + qv
            cpb = pltpu.async_copy(feat.at[gidx_b], gbuf_b, sem_b)
            cpa.wait()
            pltpu.sync_copy(gbuf_a, acc.at[scat_v.at[ja]], add=True)
            if p == 0:
                pltpu.sync_copy(ones_v, deg.at[scat_v.at[ja]], add=True)
            cpb.wait()
            pltpu.sync_copy(gbuf_b, acc.at[scat_v.at[jb]], add=True)
            if p == 0:
                pltpu.sync_copy(ones_v, deg.at[scat_v.at[jb]], add=True)
            return 0
        lax.fori_loop(0, NCH // 2, _chunk, 0)
        plsc.subcore_barrier()

        pltpu.sync_copy(acc.at[pl.ds(r0, ROWS_PER_TILE)], nbuf)
        if p == 0:
            pltpu.sync_copy(deg.at[pl.ds(r0, ROWS_PER_TILE)], degbuf)

            def _inv(r, _):
                degbuf[r, :] = 1.0 / jnp.maximum(degbuf[r, :], 1.0)
                return 0
            lax.fori_loop(0, ROWS_PER_TILE, _inv, 0)

        def _nrm(r, _):
            scale = degbuf[r, :]
            for c in range(Q // L):
                nbuf[r, pl.ds(c * L, L)] = nbuf[r, pl.ds(c * L, L)] * scale
            return 0
        lax.fori_loop(0, ROWS_PER_TILE, _nrm, 0)
        pltpu.sync_copy(nbuf, out.at[pl.ds(r0, ROWS_PER_TILE), qv])


@jax.jit
def _lightgcn_sc(hu4, srcr, dstr):
    mesh = plsc.VectorSubcoreMesh(core_axis_name="c", subcore_axis_name="s",
                                  num_cores=NC, num_subcores=NS)
    f = pl.kernel(
        _gcn_pass,
        out_type=jax.ShapeDtypeStruct((N_GROUP, NQ, Q), jnp.float32),
        mesh=mesh,
        scratch_types=[
            pltpu.VMEM((NCH, K), jnp.int32),
            pltpu.VMEM((NCH, K), jnp.int32),
            pltpu.VMEM((K,), jnp.int32),
            pltpu.VMEM((K,), jnp.int32),
            pltpu.VMEM((K, Q), jnp.float32),
            pltpu.VMEM((K, Q), jnp.float32),
            pltpu.VMEM((K, L), jnp.float32),
            pltpu.VMEM((ROWS_PER_TILE, Q), jnp.float32),
            pltpu.VMEM((ROWS_PER_TILE, L), jnp.float32),
            pltpu.VMEM_SHARED((N_GROUP, Q), jnp.float32),
            pltpu.VMEM_SHARED((N_GROUP, L), jnp.float32),
            pltpu.SemaphoreType.DMA,
            pltpu.SemaphoreType.DMA,
        ],
        compiler_params=pltpu.CompilerParams(use_tc_tiling_on_sc=False,
                                             internal_scratch_in_bytes=0),
    )
    rst4 = f(hu4, srcr, dstr)
    bsrc4 = f(rst4.reshape(NQ * N_GROUP, Q), dstr, srcr)
    rst = rst4.reshape(N_GROUP, D)
    bsrc = bsrc4.reshape(N_USER, D)
    return bsrc, rst


def kernel(h_user, h_group, edge_index):
    del h_group
    hu4 = h_user.reshape(NQ * N_USER, Q)
    src = edge_index[0].astype(jnp.int32).reshape(NS, NCH, K)
    dst = edge_index[1].astype(jnp.int32).reshape(NS, NCH, K)
    bsrc, rst = _lightgcn_sc(hu4, src, dst)
    return (bsrc, rst)


# speedup vs baseline: 5.8460x; 1.2289x over previous
import jax
import jax.numpy as jnp
from jax import lax
from jax.experimental import pallas as pl
from jax.experimental.pallas import tpu as pltpu
from jax.experimental.pallas import tpu_sc as plsc

N_USER = 10000
N_GROUP = 10000
N_EDGE = 320000
D = 128
Q = 32
NQ = D // Q
NC = 2
NS = 16
L = 16
NB = 5

EDGES_PER_TILE = N_EDGE // NS
K = 80
NCH = EDGES_PER_TILE // K
ROWS_PER_TILE = N_GROUP // NS


def _body(hu4, srcr, dstr, rst_out, bsrc_out,
          src_v, dst_v, gidx5, gbuf5, ones_v, nbuf, degbuf, acc, deg,
          sem0, sem1, sem2, sem3, sem4):
    sems = (sem0, sem1, sem2, sem3, sem4)
    cid = lax.axis_index("c")
    sid = lax.axis_index("s")
    r0 = sid * ROWS_PER_TILE

    pltpu.sync_copy(srcr.at[sid], src_v)
    pltpu.sync_copy(dstr.at[sid], dst_v)

    one16 = jnp.full((L,), 1.0, jnp.float32)
    zero16 = jnp.zeros((L,), jnp.float32)

    def _fill(r, _):
        ones_v[r, :] = one16
        return 0
    lax.fori_loop(0, K, _fill, 0)

    def _zacc(r, _):
        for c in range(Q // L):
            nbuf[r, pl.ds(c * L, L)] = zero16
        return 0

    def _zdeg(r, _):
        degbuf[r, :] = zero16
        return 0

    def _inv(r, _):
        degbuf[r, :] = 1.0 / jnp.maximum(degbuf[r, :], 1.0)
        return 0

    def _nrm(r, _):
        scale = degbuf[r, :]
        for c in range(Q // L):
            nbuf[r, pl.ds(c * L, L)] = nbuf[r, pl.ds(c * L, L)] * scale
        return 0

    def _direction(feat, gath, scat, out):
        for p in range(NQ // NC):
            qv = cid * (NQ // NC) + p
            base = qv * N_GROUP

            lax.fori_loop(0, ROWS_PER_TILE, _zacc, 0)
            pltpu.sync_copy(nbuf, acc.at[pl.ds(r0, ROWS_PER_TILE)])
            if p == 0:
                lax.fori_loop(0, ROWS_PER_TILE, _zdeg, 0)
                pltpu.sync_copy(degbuf, deg.at[pl.ds(r0, ROWS_PER_TILE)])
            plsc.subcore_barrier()

            def _chunk(j5, _):
                j0 = j5 * NB
                cps = []
                for t in range(NB):
                    jt = j0 + t
                    for k in range(K // L):
                        g16 = gath[jt, pl.ds(k * L, L)]
                        gidx5[t, pl.ds(k * L, L)] = g16 + base
                    cps.append(pltpu.async_copy(feat.at[gidx5.at[t]],
                                                gbuf5.at[t], sems[t]))
                for t in range(NB):
                    jt = j0 + t
                    cps[t].wait()
                    pltpu.sync_copy(gbuf5.at[t], acc.at[scat.at[jt]],
                                    add=True)
                    if p == 0:
                        pltpu.sync_copy(ones_v, deg.at[scat.at[jt]],
                                        add=True)
                return 0
            lax.fori_loop(0, NCH // NB, _chunk, 0)
            plsc.subcore_barrier()

            pltpu.sync_copy(acc.at[pl.ds(r0, ROWS_PER_TILE)], nbuf)
            if p == 0:
                pltpu.sync_copy(deg.at[pl.ds(r0, ROWS_PER_TILE)], degbuf)
                lax.fori_loop(0, ROWS_PER_TILE, _inv, 0)
            lax.fori_loop(0, ROWS_PER_TILE, _nrm, 0)
            pltpu.sync_copy(nbuf, out.at[pl.ds(base + r0, ROWS_PER_TILE)])

    _direction(hu4, src_v, dst_v, rst_out)
    _direction(rst_out, dst_v, src_v, bsrc_out)


@jax.jit
def _lightgcn_sc(hu4, srcr, dstr):
    mesh = plsc.VectorSubcoreMesh(core_axis_name="c", subcore_axis_name="s",
                                  num_cores=NC, num_subcores=NS)
    f = pl.kernel(
        _body,
        out_type=[
            jax.ShapeDtypeStruct((NQ * N_GROUP, Q), jnp.float32),
            jax.ShapeDtypeStruct((NQ * N_USER, Q), jnp.float32),
        ],
        mesh=mesh,
        scratch_types=[
            pltpu.VMEM((NCH, K), jnp.int32),
            pltpu.VMEM((NCH, K), jnp.int32),
            pltpu.VMEM((NB, K), jnp.int32),
            pltpu.VMEM((NB, K, Q), jnp.float32),
            pltpu.VMEM((K, L), jnp.float32),
            pltpu.VMEM((ROWS_PER_TILE, Q), jnp.float32),
            pltpu.VMEM((ROWS_PER_TILE, L), jnp.float32),
            pltpu.VMEM_SHARED((N_GROUP, Q), jnp.float32),
            pltpu.VMEM_SHARED((N_GROUP, L), jnp.float32),
            pltpu.SemaphoreType.DMA,
            pltpu.SemaphoreType.DMA,
            pltpu.SemaphoreType.DMA,
            pltpu.SemaphoreType.DMA,
            pltpu.SemaphoreType.DMA,
        ],
        compiler_params=pltpu.CompilerParams(use_tc_tiling_on_sc=False,
                                             internal_scratch_in_bytes=0),
    )
    rst16, bsrc16 = f(hu4, srcr, dstr)
    rst = rst16.reshape(NQ, N_GROUP, Q).transpose(1, 0, 2).reshape(N_GROUP, D)
    bsrc = bsrc16.reshape(NQ, N_USER, Q).transpose(1, 0, 2).reshape(N_USER, D)
    return bsrc, rst


def kernel(h_user, h_group, edge_index):
    del h_group
    hu4 = h_user.reshape(N_USER, NQ, Q).transpose(1, 0, 2).reshape(
        NQ * N_USER, Q)
    src = edge_index[0].astype(jnp.int32).reshape(NS, NCH, K)
    dst = edge_index[1].astype(jnp.int32).reshape(NS, NCH, K)
    bsrc, rst = _lightgcn_sc(hu4, src, dst)
    return (bsrc, rst)


# speedup vs baseline: 6.7363x; 1.1523x over previous
import jax
import jax.numpy as jnp
from jax import lax
from jax.experimental import pallas as pl
from jax.experimental.pallas import tpu as pltpu
from jax.experimental.pallas import tpu_sc as plsc

N_USER = 10000
N_GROUP = 10000
N_EDGE = 320000
D = 128
Q = 32
NQ = D // Q
NC = 2
NS = 16
L = 16
NB = 10

EDGES_PER_TILE = N_EDGE // NS
K = 80
NCH = EDGES_PER_TILE // K
ROWS_PER_TILE = N_GROUP // NS


def _body(hu4, srcr, dstr, rst_out, bsrc_out,
          src_v, dst_v, gidx5, gbuf5, ones_v, nbuf, degbuf, acc, deg,
          sem0, sem1, sem2, sem3, sem4, sem5, sem6, sem7, sem8, sem9):
    sems = (sem0, sem1, sem2, sem3, sem4, sem5, sem6, sem7, sem8, sem9)
    cid = lax.axis_index("c")
    sid = lax.axis_index("s")
    r0 = sid * ROWS_PER_TILE

    pltpu.sync_copy(srcr.at[sid], src_v)
    pltpu.sync_copy(dstr.at[sid], dst_v)

    one16 = jnp.full((L,), 1.0, jnp.float32)
    zero16 = jnp.zeros((L,), jnp.float32)

    def _fill(r, _):
        ones_v[r, :] = one16
        return 0
    lax.fori_loop(0, K, _fill, 0)

    def _zacc(r, _):
        for c in range(Q // L):
            nbuf[r, pl.ds(c * L, L)] = zero16
        return 0

    def _zdeg(r, _):
        degbuf[r, :] = zero16
        return 0

    def _inv(r, _):
        degbuf[r, :] = 1.0 / jnp.maximum(degbuf[r, :], 1.0)
        return 0

    def _nrm(r, _):
        scale = degbuf[r, :]
        for c in range(Q // L):
            nbuf[r, pl.ds(c * L, L)] = nbuf[r, pl.ds(c * L, L)] * scale
        return 0

    def _direction(feat, gath, scat, out):
        for p in range(NQ // NC):
            qv = cid * (NQ // NC) + p
            base = qv * N_GROUP

            lax.fori_loop(0, ROWS_PER_TILE, _zacc, 0)
            pltpu.sync_copy(nbuf, acc.at[pl.ds(r0, ROWS_PER_TILE)])
            if p == 0:
                lax.fori_loop(0, ROWS_PER_TILE, _zdeg, 0)
                pltpu.sync_copy(degbuf, deg.at[pl.ds(r0, ROWS_PER_TILE)])
            plsc.subcore_barrier()

            def _chunk(j5, _):
                j0 = j5 * NB
                cps = []
                for t in range(NB):
                    jt = j0 + t
                    for k in range(K // L):
                        g16 = gath[jt, pl.ds(k * L, L)]
                        gidx5[t, pl.ds(k * L, L)] = g16 + base
                    cps.append(pltpu.async_copy(feat.at[gidx5.at[t]],
                                                gbuf5.at[t], sems[t]))
                for t in range(NB):
                    jt = j0 + t
                    cps[t].wait()
                    pltpu.sync_copy(gbuf5.at[t], acc.at[scat.at[jt]],
                                    add=True)
                    if p == 0:
                        pltpu.sync_copy(ones_v, deg.at[scat.at[jt]],
                                        add=True)
                return 0
            lax.fori_loop(0, NCH // NB, _chunk, 0)
            plsc.subcore_barrier()

            pltpu.sync_copy(acc.at[pl.ds(r0, ROWS_PER_TILE)], nbuf)
            if p == 0:
                pltpu.sync_copy(deg.at[pl.ds(r0, ROWS_PER_TILE)], degbuf)
                lax.fori_loop(0, ROWS_PER_TILE, _inv, 0)
            lax.fori_loop(0, ROWS_PER_TILE, _nrm, 0)
            pltpu.sync_copy(nbuf, out.at[pl.ds(base + r0, ROWS_PER_TILE)])

    _direction(hu4, src_v, dst_v, rst_out)
    _direction(rst_out, dst_v, src_v, bsrc_out)


@jax.jit
def _lightgcn_sc(hu4, srcr, dstr):
    mesh = plsc.VectorSubcoreMesh(core_axis_name="c", subcore_axis_name="s",
                                  num_cores=NC, num_subcores=NS)
    f = pl.kernel(
        _body,
        out_type=[
            jax.ShapeDtypeStruct((NQ * N_GROUP, Q), jnp.float32),
            jax.ShapeDtypeStruct((NQ * N_USER, Q), jnp.float32),
        ],
        mesh=mesh,
        scratch_types=[
            pltpu.VMEM((NCH, K), jnp.int32),
            pltpu.VMEM((NCH, K), jnp.int32),
            pltpu.VMEM((NB, K), jnp.int32),
            pltpu.VMEM((NB, K, Q), jnp.float32),
            pltpu.VMEM((K, L), jnp.float32),
            pltpu.VMEM((ROWS_PER_TILE, Q), jnp.float32),
            pltpu.VMEM((ROWS_PER_TILE, L), jnp.float32),
            pltpu.VMEM_SHARED((N_GROUP, Q), jnp.float32),
            pltpu.VMEM_SHARED((N_GROUP, L), jnp.float32),
            pltpu.SemaphoreType.DMA,
            pltpu.SemaphoreType.DMA,
            pltpu.SemaphoreType.DMA,
            pltpu.SemaphoreType.DMA,
            pltpu.SemaphoreType.DMA,
            pltpu.SemaphoreType.DMA,
            pltpu.SemaphoreType.DMA,
            pltpu.SemaphoreType.DMA,
            pltpu.SemaphoreType.DMA,
            pltpu.SemaphoreType.DMA,
        ],
        compiler_params=pltpu.CompilerParams(use_tc_tiling_on_sc=False,
                                             internal_scratch_in_bytes=0),
    )
    rst16, bsrc16 = f(hu4, srcr, dstr)
    rst = rst16.reshape(NQ, N_GROUP, Q).transpose(1, 0, 2).reshape(N_GROUP, D)
    bsrc = bsrc16.reshape(NQ, N_USER, Q).transpose(1, 0, 2).reshape(N_USER, D)
    return bsrc, rst


def kernel(h_user, h_group, edge_index):
    del h_group
    hu4 = h_user.reshape(N_USER, NQ, Q).transpose(1, 0, 2).reshape(
        NQ * N_USER, Q)
    src = edge_index[0].astype(jnp.int32).reshape(NS, NCH, K)
    dst = edge_index[1].astype(jnp.int32).reshape(NS, NCH, K)
    bsrc, rst = _lightgcn_sc(hu4, src, dst)
    return (bsrc, rst)


# speedup vs baseline: 7.8480x; 1.1650x over previous
import jax
import jax.numpy as jnp
from jax import lax
from jax.experimental import pallas as pl
from jax.experimental.pallas import tpu as pltpu
from jax.experimental.pallas import tpu_sc as plsc

N_USER = 10000
N_GROUP = 10000
N_EDGE = 320000
D = 128
Q = 32
NQ = D // Q
NC = 2
NS = 16
L = 16
NB = 10

EDGES_PER_TILE = N_EDGE // NS
K = 80
NCH = EDGES_PER_TILE // K
ROWS_PER_TILE = N_GROUP // NS


def _body(hu4, srcr, dstr, rst_out, bsrc_out,
          src_v, dst_v, gidx5, gbuf5, ones_v, nbuf, degbuf, acc, deg,
          semg, sems_sc, semd):
    cid = lax.axis_index("c")
    sid = lax.axis_index("s")
    r0 = sid * ROWS_PER_TILE

    pltpu.sync_copy(srcr.at[sid], src_v)
    pltpu.sync_copy(dstr.at[sid], dst_v)

    one16 = jnp.full((L,), 1.0, jnp.float32)
    zero16 = jnp.zeros((L,), jnp.float32)

    def _fill(r, _):
        ones_v[r, :] = one16
        return 0
    lax.fori_loop(0, K, _fill, 0)

    def _zacc(r, _):
        for c in range(Q // L):
            nbuf[r, pl.ds(c * L, L)] = zero16
        return 0

    def _zdeg(r, _):
        degbuf[r, :] = zero16
        return 0

    def _inv(r, _):
        degbuf[r, :] = 1.0 / jnp.maximum(degbuf[r, :], 1.0)
        return 0

    def _nrm(r, _):
        scale = degbuf[r, :]
        for c in range(Q // L):
            nbuf[r, pl.ds(c * L, L)] = nbuf[r, pl.ds(c * L, L)] * scale
        return 0

    def _direction(feat, gath, scat, out):
        for p in range(NQ // NC):
            qv = cid * (NQ // NC) + p
            base = qv * N_GROUP

            lax.fori_loop(0, ROWS_PER_TILE, _zacc, 0)
            pltpu.sync_copy(nbuf, acc.at[pl.ds(r0, ROWS_PER_TILE)])
            if p == 0:
                lax.fori_loop(0, ROWS_PER_TILE, _zdeg, 0)
                pltpu.sync_copy(degbuf, deg.at[pl.ds(r0, ROWS_PER_TILE)])
            plsc.subcore_barrier()

            def _chunk(j5, _):
                j0 = j5 * NB
                gcps = []
                for t in range(NB):
                    jt = j0 + t
                    for k in range(K // L):
                        g16 = gath[jt, pl.ds(k * L, L)]
                        gidx5[t, pl.ds(k * L, L)] = g16 + base
                    gcps.append(pltpu.async_copy(feat.at[gidx5.at[t]],
                                                 gbuf5.at[t], semg.at[t]))
                scps = []
                for t in range(NB):
                    jt = j0 + t
                    gcps[t].wait()
                    scps.append(pltpu.async_copy(gbuf5.at[t],
                                                 acc.at[scat.at[jt]],
                                                 sems_sc.at[t], add=True))
                    if p == 0:
                        scps.append(pltpu.async_copy(ones_v,
                                                     deg.at[scat.at[jt]],
                                                     semd.at[t], add=True))
                for cp in scps:
                    cp.wait()
                return 0
            lax.fori_loop(0, NCH // NB, _chunk, 0)
            plsc.subcore_barrier()

            pltpu.sync_copy(acc.at[pl.ds(r0, ROWS_PER_TILE)], nbuf)
            if p == 0:
                pltpu.sync_copy(deg.at[pl.ds(r0, ROWS_PER_TILE)], degbuf)
                lax.fori_loop(0, ROWS_PER_TILE, _inv, 0)
            lax.fori_loop(0, ROWS_PER_TILE, _nrm, 0)
            pltpu.sync_copy(nbuf, out.at[pl.ds(base + r0, ROWS_PER_TILE)])

    _direction(hu4, src_v, dst_v, rst_out)
    _direction(rst_out, dst_v, src_v, bsrc_out)


@jax.jit
def _lightgcn_sc(hu4, srcr, dstr):
    mesh = plsc.VectorSubcoreMesh(core_axis_name="c", subcore_axis_name="s",
                                  num_cores=NC, num_subcores=NS)
    f = pl.kernel(
        _body,
        out_type=[
            jax.ShapeDtypeStruct((NQ * N_GROUP, Q), jnp.float32),
            jax.ShapeDtypeStruct((NQ * N_USER, Q), jnp.float32),
        ],
        mesh=mesh,
        scratch_types=[
            pltpu.VMEM((NCH, K), jnp.int32),
            pltpu.VMEM((NCH, K), jnp.int32),
            pltpu.VMEM((NB, K), jnp.int32),
            pltpu.VMEM((NB, K, Q), jnp.float32),
            pltpu.VMEM((K, L), jnp.float32),
            pltpu.VMEM((ROWS_PER_TILE, Q), jnp.float32),
            pltpu.VMEM((ROWS_PER_TILE, L), jnp.float32),
            pltpu.VMEM_SHARED((N_GROUP, Q), jnp.float32),
            pltpu.VMEM_SHARED((N_GROUP, L), jnp.float32),
            pltpu.SemaphoreType.DMA((NB,)),
            pltpu.SemaphoreType.DMA((NB,)),
            pltpu.SemaphoreType.DMA((NB,)),
        ],
        compiler_params=pltpu.CompilerParams(use_tc_tiling_on_sc=False,
                                             internal_scratch_in_bytes=0),
    )
    rst16, bsrc16 = f(hu4, srcr, dstr)
    rst = rst16.reshape(NQ, N_GROUP, Q).transpose(1, 0, 2).reshape(N_GROUP, D)
    bsrc = bsrc16.reshape(NQ, N_USER, Q).transpose(1, 0, 2).reshape(N_USER, D)
    return bsrc, rst


def kernel(h_user, h_group, edge_index):
    del h_group
    hu4 = h_user.reshape(N_USER, NQ, Q).transpose(1, 0, 2).reshape(
        NQ * N_USER, Q)
    src = edge_index[0].astype(jnp.int32).reshape(NS, NCH, K)
    dst = edge_index[1].astype(jnp.int32).reshape(NS, NCH, K)
    bsrc, rst = _lightgcn_sc(hu4, src, dst)
    return (bsrc, rst)
